# baseline (speedup 1.0000x reference)
import math

import jax
import jax.numpy as jnp
from jax import lax
from jax.experimental import pallas as pl

N = 1024
V = 100000
PAD = 0
EPS = 0.1 / (V - 1)
CONF = 0.9
C1 = (V - 2) * EPS * math.log(EPS) + CONF * math.log(CONF)

_BR = 16
_NB = N // _BR


def _tc_body_full(x_ref, t_ref, out_ref):
    j = pl.program_id(0)

    @pl.when(j == 0)
    def _init():
        out_ref[...] = jnp.zeros((1, 1), jnp.float32)

    tgt = t_ref[...]
    nonpad = (tgt != PAD).astype(jnp.float32)
    col = lax.broadcasted_iota(jnp.int32, (1, V), 1)
    xb = x_ref[...]
    rowsum = jnp.sum(xb, axis=1, keepdims=True)
    g_row = jnp.sum(jnp.where(col == tgt, xb, 0.0), axis=1, keepdims=True)
    per_row = C1 + EPS * xb[:, 0:1] - EPS * rowsum - (CONF - EPS) * g_row
    out_ref[...] += jnp.sum(per_row * nonpad).reshape(1, 1)


def kernel(x, target):
    tgt = target.astype(jnp.int32)
    loss = pl.pallas_call(
        _tc_body_full,
        grid=(_NB,),
        in_specs=[
            pl.BlockSpec((_BR, V), lambda j: (j, 0)),
            pl.BlockSpec((_BR, 1), lambda j: (j, 0)),
        ],
        out_specs=pl.BlockSpec((1, 1), lambda j: (0, 0)),
        out_shape=jax.ShapeDtypeStruct((1, 1), jnp.float32),
    )(x, tgt.reshape(N, 1))
    return jnp.reshape(loss, ())


# speedup vs baseline: 1.0643x; 1.0643x over previous
import math

import jax
import jax.numpy as jnp
from jax import lax
from jax.experimental import pallas as pl

N = 1024
V = 100000
PAD = 0
EPS = 0.1 / (V - 1)
CONF = 0.9
C1 = (V - 2) * EPS * math.log(EPS) + CONF * math.log(CONF)

_BR = 16
_NS = 4
_NB = N // (_BR * _NS)


def _tc_body_full(*refs):
    out_ref = refs[-1]
    x_refs = refs[:_NS]
    t_refs = refs[_NS:2 * _NS]
    j = pl.program_id(0)

    @pl.when(j == 0)
    def _init():
        out_ref[...] = jnp.zeros((1, 1), jnp.float32)

    col = lax.broadcasted_iota(jnp.int32, (1, V), 1)
    acc = jnp.float32(0.0)
    for x_ref, t_ref in zip(x_refs, t_refs):
        tgt = t_ref[...]
        nonpad = (tgt != PAD).astype(jnp.float32)
        xb = x_ref[...]
        rowsum = jnp.sum(xb, axis=1, keepdims=True)
        g_row = jnp.sum(jnp.where(col == tgt, xb, 0.0), axis=1, keepdims=True)
        per_row = C1 + EPS * xb[:, 0:1] - EPS * rowsum - (CONF - EPS) * g_row
        acc += jnp.sum(per_row * nonpad)
    out_ref[...] += acc.reshape(1, 1)


def kernel(x, target):
    tgt = target.astype(jnp.int32)
    x_specs = [
        pl.BlockSpec((_BR, V), lambda j, q=q: (_NB * q + j, 0))
        for q in range(_NS)
    ]
    t_specs = [
        pl.BlockSpec((_BR, 1), lambda j, q=q: (_NB * q + j, 0))
        for q in range(_NS)
    ]
    loss = pl.pallas_call(
        _tc_body_full,
        grid=(_NB,),
        in_specs=x_specs + t_specs,
        out_specs=pl.BlockSpec((1, 1), lambda j: (0, 0)),
        out_shape=jax.ShapeDtypeStruct((1, 1), jnp.float32),
    )(*([x] * _NS + [tgt.reshape(N, 1)] * _NS))
    return jnp.reshape(loss, ())


# speedup vs baseline: 1.0688x; 1.0041x over previous
import math

import jax
import jax.numpy as jnp
from jax import lax
from jax.experimental import pallas as pl
from jax.experimental.pallas import tpu as pltpu

N = 1024
V = 100000
PAD = 0
EPS = 0.1 / (V - 1)
CONF = 0.9
C1 = (V - 2) * EPS * math.log(EPS) + CONF * math.log(CONF)

_BR = 16
_NCH = N // _BR
_K = 6


def _body(x_hbm, t_ref, out_ref, buf, sems):
    def start(c):
        pltpu.make_async_copy(
            x_hbm.at[pl.ds(c * _BR, _BR), :],
            buf.at[pl.ds((c % _K) * _BR, _BR), :],
            sems.at[c % _K],
        ).start()

    def wait(c):
        pltpu.make_async_copy(
            x_hbm.at[pl.ds(c * _BR, _BR), :],
            buf.at[pl.ds((c % _K) * _BR, _BR), :],
            sems.at[c % _K],
        ).wait()

    for c in range(_K):
        start(c)

    col = lax.broadcasted_iota(jnp.int32, (1, V), 1)

    def step(c, acc):
        wait(c)
        xb = buf[pl.ds((c % _K) * _BR, _BR), :]
        tgt = t_ref[pl.ds(c * _BR, _BR), :]
        nonpad = (tgt != PAD).astype(jnp.float32)
        rowsum = jnp.sum(xb, axis=1, keepdims=True)
        g_row = jnp.sum(jnp.where(col == tgt, xb, 0.0), axis=1, keepdims=True)
        per_row = C1 + EPS * xb[:, 0:1] - EPS * rowsum - (CONF - EPS) * g_row
        acc += jnp.sum(per_row * nonpad)

        @pl.when(c + _K < _NCH)
        def _():
            start(c + _K)

        return acc

    acc = lax.fori_loop(0, _NCH, step, jnp.float32(0.0))
    out_ref[...] = acc.reshape(1, 1)


def kernel(x, target):
    tgt = target.astype(jnp.int32)
    loss = pl.pallas_call(
        _body,
        in_specs=[
            pl.BlockSpec(memory_space=pltpu.HBM),
            pl.BlockSpec(memory_space=pltpu.VMEM),
        ],
        out_specs=pl.BlockSpec(memory_space=pltpu.VMEM),
        out_shape=jax.ShapeDtypeStruct((1, 1), jnp.float32),
        scratch_shapes=[
            pltpu.VMEM((_K * _BR, V), jnp.float32),
            pltpu.SemaphoreType.DMA((_K,)),
        ],
    )(x, tgt.reshape(N, 1))
    return jnp.reshape(loss, ())
